# baseline (speedup 1.0000x reference)
import functools

import jax
import jax.numpy as jnp
from jax import lax
from jax.experimental import pallas as pl
from jax.experimental.pallas import tpu as pltpu
from jax.experimental.pallas import tpu_sc as plsc

N = 100000
IN_DIM = 256
HIDDEN = 512
K = 10000
R = 1024
GRID = 98
NPAD = R * GRID
NW = 32
CHUNK = NPAD // NW
BINS = 512
MPAD = 10240
SLOTS = MPAD // NW
KSEL = K + 8
NEG_INF = float("-inf")
POS_INF = float("inf")


@functools.cache
def _mesh():
    return plsc.VectorSubcoreMesh(core_axis_name="c", subcore_axis_name="s",
                                  num_cores=2, num_subcores=16)


def _mlp_body(x_ref, w1_ref, b1_ref, w2_ref, b2_ref,
              logits_ref, m_ref, s_ref, mn_ref, m_acc, s_acc, mn_acc):
    i = pl.program_id(0)
    x = x_ref[...]
    h = jnp.dot(x, w1_ref[...], preferred_element_type=jnp.float32)
    h = jnp.maximum(h + b1_ref[...], 0.0)
    hw = h * w2_ref[...]
    logit = jnp.sum(jnp.reshape(hw, (8, 128, HIDDEN)), axis=-1) + b2_ref[...]
    row = (i * R
           + jax.lax.broadcasted_iota(jnp.int32, (8, 128), 0) * 128
           + jax.lax.broadcasted_iota(jnp.int32, (8, 128), 1))
    valid = row < N
    logit = jnp.where(valid, logit, NEG_INF)
    logits_ref[0] = logit

    @pl.when(i == 0)
    def _init():
        m_acc[...] = jnp.full((8, 128), NEG_INF, jnp.float32)
        s_acc[...] = jnp.zeros((8, 128), jnp.float32)
        mn_acc[...] = jnp.full((8, 128), POS_INF, jnp.float32)

    m_old = m_acc[...]
    m_new = jnp.maximum(m_old, logit)
    scale = jnp.where(m_old == NEG_INF, 0.0, jnp.exp(m_old - m_new))
    term = jnp.where(logit == NEG_INF, 0.0, jnp.exp(logit - m_new))
    s_new = s_acc[...] * scale + term
    m_acc[...] = m_new
    s_acc[...] = s_new
    mn_new = jnp.minimum(mn_acc[...], jnp.where(valid, logit, POS_INF))
    mn_acc[...] = mn_new

    @pl.when(i == GRID - 1)
    def _fin():
        m_all = jnp.max(m_new)
        s_all = jnp.sum(s_new * jnp.exp(m_new - m_all))
        m_ref[...] = jnp.reshape(m_all, (1, 1))
        s_ref[...] = jnp.reshape(s_all, (1, 1))
        mn_ref[...] = jnp.reshape(jnp.min(mn_new), (1, 1))


def _mlp_logits(xp, W1, b1r, w2r, b2r):
    return pl.pallas_call(
        _mlp_body,
        grid=(GRID,),
        in_specs=[
            pl.BlockSpec((R, IN_DIM), lambda i: (i, 0)),
            pl.BlockSpec((IN_DIM, HIDDEN), lambda i: (0, 0)),
            pl.BlockSpec((1, HIDDEN), lambda i: (0, 0)),
            pl.BlockSpec((1, HIDDEN), lambda i: (0, 0)),
            pl.BlockSpec((1, 1), lambda i: (0, 0)),
        ],
        out_specs=[
            pl.BlockSpec((1, 8, 128), lambda i: (i, 0, 0)),
            pl.BlockSpec((1, 1), lambda i: (0, 0)),
            pl.BlockSpec((1, 1), lambda i: (0, 0)),
            pl.BlockSpec((1, 1), lambda i: (0, 0)),
        ],
        out_shape=[
            jax.ShapeDtypeStruct((GRID, 8, 128), jnp.float32),
            jax.ShapeDtypeStruct((1, 1), jnp.float32),
            jax.ShapeDtypeStruct((1, 1), jnp.float32),
            jax.ShapeDtypeStruct((1, 1), jnp.float32),
        ],
        scratch_shapes=[
            pltpu.VMEM((8, 128), jnp.float32),
            pltpu.VMEM((8, 128), jnp.float32),
            pltpu.VMEM((8, 128), jnp.float32),
        ],
    )(xp, W1, b1r, w2r, b2r)


def _suffix_body(v_ref, edges_ref, out_ref, acc):
    i = pl.program_id(0)

    @pl.when(i == 0)
    def _init():
        acc[...] = jnp.zeros((1, BINS), jnp.float32)

    v = v_ref[...]
    ge = jnp.where(v >= edges_ref[...], 1.0, 0.0)
    acc[...] += jnp.sum(ge, axis=0, keepdims=True)

    @pl.when(i == GRID - 1)
    def _fin():
        out_ref[...] = acc[...]


def _suffix_counts(logits_col, edges):
    return pl.pallas_call(
        _suffix_body,
        grid=(GRID,),
        in_specs=[
            pl.BlockSpec((R, 1), lambda i: (i, 0)),
            pl.BlockSpec((1, BINS), lambda i: (0, 0)),
        ],
        out_specs=pl.BlockSpec((1, BINS), lambda i: (0, 0)),
        out_shape=jax.ShapeDtypeStruct((1, BINS), jnp.float32),
        scratch_shapes=[pltpu.VMEM((1, BINS), jnp.float32)],
    )(logits_col, edges)


def _dest_body(v_ref, t_ref, dest_ref, tot_ref, carry):
    i = pl.program_id(0)

    @pl.when(i == 0)
    def _init():
        carry[...] = jnp.zeros((1, 1), jnp.float32)

    v = v_ref[...]
    mask = jnp.where(v >= t_ref[...], 1.0, 0.0)
    rr = jax.lax.broadcasted_iota(jnp.int32, (R, R), 0)
    cc = jax.lax.broadcasted_iota(jnp.int32, (R, R), 1)
    tri = jnp.where(rr >= cc, 1.0, 0.0)
    prefix = jnp.dot(tri, mask, preferred_element_type=jnp.float32)
    dest = carry[...] + prefix - 1.0
    dest_i = jnp.where(mask > 0.0, dest.astype(jnp.int32), -1)
    dest_ref[...] = dest_i
    carry_new = carry[...] + jnp.sum(mask, keepdims=True)
    carry[...] = carry_new

    @pl.when(i == GRID - 1)
    def _fin():
        tot_ref[...] = carry_new.astype(jnp.int32)


def _dests(logits_col, t_edge):
    return pl.pallas_call(
        _dest_body,
        grid=(GRID,),
        in_specs=[
            pl.BlockSpec((R, 1), lambda i: (i, 0)),
            pl.BlockSpec((1, 1), lambda i: (0, 0)),
        ],
        out_specs=[
            pl.BlockSpec((R, 1), lambda i: (i, 0)),
            pl.BlockSpec((1, 1), lambda i: (0, 0)),
        ],
        out_shape=[
            jax.ShapeDtypeStruct((NPAD, 1), jnp.int32),
            jax.ShapeDtypeStruct((1, 1), jnp.int32),
        ],
        scratch_shapes=[pltpu.VMEM((1, 1), jnp.float32)],
    )(logits_col, t_edge)


_CBLK = [128] * 24 + [64]


def _compact_body(logits_hbm, dests_hbm, cvals_hbm, cidx_hbm, *scr):
    chunk_v, gidx_v, dchunk_v = scr[0], scr[1], scr[2]
    dls = scr[3:3 + len(_CBLK)]
    sem = scr[-1]
    c = lax.axis_index("c")
    s = lax.axis_index("s")
    wid = s * 2 + c
    lanes = lax.iota(jnp.int32, 16)
    pltpu.sync_copy(logits_hbm.at[pl.ds(wid * CHUNK, CHUNK)], chunk_v)
    pltpu.sync_copy(dests_hbm.at[pl.ds(wid * CHUNK, CHUNK)], dchunk_v)

    for i in range(CHUNK // 16):
        j = i * 16 + lanes
        gidx_v[pl.ds(i * 16, 16)] = wid * CHUNK + j
        d = dchunk_v[pl.ds(i * 16, 16)]
        ok = (d >= 0) & (d < MPAD)
        dd = jnp.where(ok, d, MPAD + (j & 63))
        blk, off = i // 8, (i % 8) * 16
        dls[blk][pl.ds(off, 16)] = dd

    copies = []
    pos = 0
    for blk, blen in enumerate(_CBLK):
        copies.append(pltpu.async_copy(
            chunk_v.at[pl.ds(pos, blen)], cvals_hbm.at[dls[blk]], sem))
        copies.append(pltpu.async_copy(
            gidx_v.at[pl.ds(pos, blen)], cidx_hbm.at[dls[blk]], sem))
        pos += blen
    for cp in copies:
        cp.wait()


@functools.cache
def _compact_kernel():
    return pl.kernel(
        _compact_body,
        out_type=[
            jax.ShapeDtypeStruct((MPAD + 64,), jnp.float32),
            jax.ShapeDtypeStruct((MPAD + 64,), jnp.int32),
        ],
        mesh=_mesh(),
        scratch_types=[
            pltpu.VMEM((CHUNK,), jnp.float32),
            pltpu.VMEM((CHUNK,), jnp.int32),
            pltpu.VMEM((CHUNK,), jnp.int32),
        ] + [pltpu.VMEM((blen,), jnp.int32) for blen in _CBLK] + [
            pltpu.SemaphoreType.DMA,
        ],
    )


JCH = 1024


def _rank_body(vi_ref, vj_ref, ii_ref, jj_ref, m_ref, s_ref, mt_ref,
               rank_ref, prob_ref):
    g = pl.program_id(0)
    vi = vi_ref[...]
    ii = ii_ref[...]
    vj = vj_ref[...]
    jj = jj_ref[...]
    mtot = mt_ref[...]
    ipos = g * JCH + jax.lax.broadcasted_iota(jnp.int32, (JCH, 1), 0)
    jpos_full = jax.lax.broadcasted_iota(jnp.int32, (1, MPAD), 1)

    acc = jnp.zeros((JCH, 1), jnp.float32)
    for cc in range(MPAD // JCH):
        vjc = vj[:, cc * JCH:(cc + 1) * JCH]
        jjc = jj[:, cc * JCH:(cc + 1) * JCH]
        jpc = jpos_full[:, cc * JCH:(cc + 1) * JCH]
        jvalid = jpc < mtot
        before = (vjc > vi) | ((vjc == vi) & (jjc < ii))
        cnt = jnp.where(before & jvalid, 1.0, 0.0)
        acc = acc + jnp.sum(cnt, axis=1, keepdims=True)

    ivalid = ipos < mtot
    rank = jnp.where(ivalid, acc.astype(jnp.int32), MPAD + ipos)
    rank_ref[...] = rank
    p = jnp.exp(vi - m_ref[...]) / s_ref[...]
    prob_ref[...] = jnp.where(ivalid, p, 0.0)


def _rank(vals_col, vals_row, idx_col, idx_row, m, s, mtot):
    return pl.pallas_call(
        _rank_body,
        grid=(MPAD // JCH,),
        in_specs=[
            pl.BlockSpec((JCH, 1), lambda g: (g, 0)),
            pl.BlockSpec((1, MPAD), lambda g: (0, 0)),
            pl.BlockSpec((JCH, 1), lambda g: (g, 0)),
            pl.BlockSpec((1, MPAD), lambda g: (0, 0)),
            pl.BlockSpec((1, 1), lambda g: (0, 0)),
            pl.BlockSpec((1, 1), lambda g: (0, 0)),
            pl.BlockSpec((1, 1), lambda g: (0, 0)),
        ],
        out_specs=[
            pl.BlockSpec((JCH, 1), lambda g: (g, 0)),
            pl.BlockSpec((JCH, 1), lambda g: (g, 0)),
        ],
        out_shape=[
            jax.ShapeDtypeStruct((MPAD, 1), jnp.int32),
            jax.ShapeDtypeStruct((MPAD, 1), jnp.float32),
        ],
    )(vals_col, vals_row, idx_col, idx_row, m, s, mtot)


OUTB = 10048
TRASH0 = K + 8


def _final_body(ranks_hbm, probs_hbm, cidx_hbm, loc_hbm,
                outbuf_hbm, partials_hbm,
                rk_v, pb_v, ci_v, sd0, sd1, sd2, cbuf_v, pt_v, sem):
    c = lax.axis_index("c")
    s = lax.axis_index("s")
    wid = s * 2 + c
    lanes = lax.iota(jnp.int32, 16)
    onesv = jnp.ones((16,), jnp.int32)
    base = wid * SLOTS
    pltpu.sync_copy(ranks_hbm.at[pl.ds(base, SLOTS)], rk_v)
    pltpu.sync_copy(probs_hbm.at[pl.ds(base, SLOTS)], pb_v)
    pltpu.sync_copy(cidx_hbm.at[pl.ds(base, SLOTS)], ci_v)

    sds = (sd0, sd0, sd0, sd0, sd0, sd0, sd0, sd0,
           sd1, sd1, sd1, sd1, sd1, sd1, sd1, sd1,
           sd2, sd2, sd2, sd2)
    offs = (0, 1, 2, 3, 4, 5, 6, 7, 0, 1, 2, 3, 4, 5, 6, 7, 0, 1, 2, 3)

    def _fire3(mk_idx, launch):
        for i in range(SLOTS // 16):
            sds[i][pl.ds(offs[i] * 16, 16)] = mk_idx(i)
        copies = [launch(sd0, 0, 128), launch(sd1, 128, 128),
                  launch(sd2, 256, 64)]
        for cp in copies:
            cp.wait()

    def _rank_dest(i):
        r = rk_v[pl.ds(i * 16, 16)]
        trash = TRASH0 + ((onesv * wid + i) & 31)
        return jnp.where(r < K, r, trash)
    _fire3(_rank_dest,
           lambda sd, pos, n: pltpu.async_copy(
               pb_v.at[pl.ds(pos, n)], outbuf_hbm.at[sd], sem))

    maccs = []
    haccs = []
    for c3 in range(3):
        def _loc_idx(i, c3=c3):
            r = rk_v[pl.ds(i * 16, 16)]
            idxc = jnp.clip(ci_v[pl.ds(i * 16, 16)], 0, N - 1)
            return jnp.where(r < K, idxc * 3 + c3, 3 * N + c3)
        _fire3(_loc_idx,
               lambda sd, pos, n: pltpu.async_copy(
                   loc_hbm.at[sd], cbuf_v.at[pl.ds(pos, n)], sem))
        macc = jnp.zeros((16,), jnp.float32)
        for i in range(SLOTS // 16):
            macc = macc + cbuf_v[pl.ds(i * 16, 16)]
        maccs.append(macc)

        def _hi_idx(i, c3=c3):
            r = rk_v[pl.ds(i * 16, 16)]
            idxc = jnp.clip(ci_v[pl.ds(i * 16, 16)], 0, N - 1)
            return jnp.where(r == 0, idxc * 3 + c3, 3 * N + c3)
        _fire3(_hi_idx,
               lambda sd, pos, n: pltpu.async_copy(
                   loc_hbm.at[sd], cbuf_v.at[pl.ds(pos, n)], sem))
        hacc = jnp.zeros((16,), jnp.float32)
        for i in range(SLOTS // 16):
            hacc = hacc + cbuf_v[pl.ds(i * 16, 16)]
        haccs.append(hacc)

    for c3 in range(3):
        pt_v[...] = maccs[c3]
        pltpu.sync_copy(pt_v, partials_hbm.at[pl.ds((wid * 6 + c3) * 16, 16)])
        pt_v[...] = haccs[c3]
        pltpu.sync_copy(
            pt_v, partials_hbm.at[pl.ds((wid * 6 + 3 + c3) * 16, 16)])


@functools.cache
def _final_kernel():
    return pl.kernel(
        _final_body,
        out_type=[
            jax.ShapeDtypeStruct((OUTB,), jnp.float32),
            jax.ShapeDtypeStruct((NW * 6 * 16,), jnp.float32),
        ],
        mesh=_mesh(),
        scratch_types=[
            pltpu.VMEM((SLOTS,), jnp.int32),
            pltpu.VMEM((SLOTS,), jnp.float32),
            pltpu.VMEM((SLOTS,), jnp.int32),
            pltpu.VMEM((128,), jnp.int32),
            pltpu.VMEM((128,), jnp.int32),
            pltpu.VMEM((64,), jnp.int32),
            pltpu.VMEM((SLOTS,), jnp.float32),
            pltpu.VMEM((16,), jnp.float32),
            pltpu.SemaphoreType.DMA,
        ],
    )


def kernel(point_features, point_locations, W1, b1, W2, b2):
    xp = jnp.pad(point_features, ((0, NPAD - N), (0, 0)))
    b1r = b1.reshape(1, HIDDEN)
    w2r = W2.reshape(1, HIDDEN)
    b2r = b2.reshape(1, 1)
    logits3, m, s, mn = _mlp_logits(xp, W1, b1r, w2r, b2r)
    logits = logits3.reshape(NPAD)
    logits_col = logits3.reshape(NPAD, 1)

    ar = jnp.arange(BINS, dtype=jnp.int32)
    arf = ar.astype(jnp.float32)
    width1 = (m[0, 0] - mn[0, 0]) / BINS
    edges1 = (mn[0, 0] + arf * width1).reshape(1, BINS)
    suf1 = _suffix_counts(logits_col, edges1)[0]
    b1i = jnp.max(jnp.where(suf1 >= KSEL, ar, 0))
    t1 = edges1[0, b1i]

    width2 = width1 / BINS
    edges2 = (t1 + arf * width2).reshape(1, BINS)
    suf2 = _suffix_counts(logits_col, edges2)[0]
    b2i = jnp.max(jnp.where(suf2 >= KSEL, ar, 0))
    t_edge = edges2[0, b2i].reshape(1, 1)

    dests, mtot = _dests(logits_col, t_edge)
    mtot = jnp.minimum(mtot, MPAD)

    cvals, cidx = _compact_kernel()(logits, dests.reshape(NPAD))
    vals_m = cvals[:MPAD]
    idx_m = cidx[:MPAD]
    ranks, probs = _rank(vals_m.reshape(MPAD, 1), vals_m.reshape(1, MPAD),
                         idx_m.reshape(MPAD, 1), idx_m.reshape(1, MPAD),
                         m, s, mtot)

    locflat = jnp.pad(point_locations, ((0, 8), (0, 0))).reshape(-1)
    outbuf, partials = _final_kernel()(ranks.reshape(MPAD),
                                       probs.reshape(MPAD), idx_m, locflat)
    pmat = partials.reshape(NW, 6, 16).sum(axis=(0, 2))
    mean_location = pmat[0:3] / K
    highest = pmat[3:6]
    return jnp.concatenate([outbuf[:K], mean_location, highest], axis=0)


# speedup vs baseline: 17.6437x; 17.6437x over previous
import functools

import jax
import jax.numpy as jnp
from jax import lax
from jax.experimental import pallas as pl
from jax.experimental.pallas import tpu as pltpu
from jax.experimental.pallas import tpu_sc as plsc

N = 100000
IN_DIM = 256
HIDDEN = 512
K = 10000
R = 1024
GRID = 98
NPAD = R * GRID
NW = 32
CHUNK = NPAD // NW
BINS = 512
MPAD = 10240
SLOTS = MPAD // NW
KSEL = K + 8
NEG_INF = float("-inf")
POS_INF = float("inf")


@functools.cache
def _mesh():
    return plsc.VectorSubcoreMesh(core_axis_name="c", subcore_axis_name="s",
                                  num_cores=2, num_subcores=16)


def _mlp_body(x_ref, w1_ref, b1_ref, w2_ref, b2_ref,
              logits_ref, m_ref, s_ref, mn_ref, m_acc, s_acc, mn_acc):
    i = pl.program_id(0)
    x = x_ref[...]
    h = jnp.dot(x, w1_ref[...], preferred_element_type=jnp.float32)
    h = jnp.maximum(h + b1_ref[...], 0.0)
    hw = h * w2_ref[...]
    logit = jnp.sum(jnp.reshape(hw, (8, 128, HIDDEN)), axis=-1) + b2_ref[...]
    row = (i * R
           + jax.lax.broadcasted_iota(jnp.int32, (8, 128), 0) * 128
           + jax.lax.broadcasted_iota(jnp.int32, (8, 128), 1))
    valid = row < N
    logit = jnp.where(valid, logit, NEG_INF)
    logits_ref[0] = logit

    @pl.when(i == 0)
    def _init():
        m_acc[...] = jnp.full((8, 128), NEG_INF, jnp.float32)
        s_acc[...] = jnp.zeros((8, 128), jnp.float32)
        mn_acc[...] = jnp.full((8, 128), POS_INF, jnp.float32)

    m_old = m_acc[...]
    m_new = jnp.maximum(m_old, logit)
    scale = jnp.where(m_old == NEG_INF, 0.0, jnp.exp(m_old - m_new))
    term = jnp.where(logit == NEG_INF, 0.0, jnp.exp(logit - m_new))
    s_new = s_acc[...] * scale + term
    m_acc[...] = m_new
    s_acc[...] = s_new
    mn_new = jnp.minimum(mn_acc[...], jnp.where(valid, logit, POS_INF))
    mn_acc[...] = mn_new

    @pl.when(i == GRID - 1)
    def _fin():
        m_all = jnp.max(m_new)
        s_all = jnp.sum(s_new * jnp.exp(m_new - m_all))
        m_ref[...] = jnp.reshape(m_all, (1, 1))
        s_ref[...] = jnp.reshape(s_all, (1, 1))
        mn_ref[...] = jnp.reshape(jnp.min(mn_new), (1, 1))


def _mlp_logits(xp, W1, b1r, w2r, b2r):
    return pl.pallas_call(
        _mlp_body,
        grid=(GRID,),
        in_specs=[
            pl.BlockSpec((R, IN_DIM), lambda i: (i, 0)),
            pl.BlockSpec((IN_DIM, HIDDEN), lambda i: (0, 0)),
            pl.BlockSpec((1, HIDDEN), lambda i: (0, 0)),
            pl.BlockSpec((1, HIDDEN), lambda i: (0, 0)),
            pl.BlockSpec((1, 1), lambda i: (0, 0)),
        ],
        out_specs=[
            pl.BlockSpec((1, 8, 128), lambda i: (i, 0, 0)),
            pl.BlockSpec((1, 1), lambda i: (0, 0)),
            pl.BlockSpec((1, 1), lambda i: (0, 0)),
            pl.BlockSpec((1, 1), lambda i: (0, 0)),
        ],
        out_shape=[
            jax.ShapeDtypeStruct((GRID, 8, 128), jnp.float32),
            jax.ShapeDtypeStruct((1, 1), jnp.float32),
            jax.ShapeDtypeStruct((1, 1), jnp.float32),
            jax.ShapeDtypeStruct((1, 1), jnp.float32),
        ],
        scratch_shapes=[
            pltpu.VMEM((8, 128), jnp.float32),
            pltpu.VMEM((8, 128), jnp.float32),
            pltpu.VMEM((8, 128), jnp.float32),
        ],
    )(xp, W1, b1r, w2r, b2r)


def _suffix_body(v_ref, edges_ref, out_ref, acc):
    i = pl.program_id(0)

    @pl.when(i == 0)
    def _init():
        acc[...] = jnp.zeros((1, BINS), jnp.float32)

    v = v_ref[...]
    ge = jnp.where(v >= edges_ref[...], 1.0, 0.0)
    acc[...] += jnp.sum(ge, axis=0, keepdims=True)

    @pl.when(i == GRID - 1)
    def _fin():
        out_ref[...] = acc[...]


def _suffix_counts(logits_col, edges):
    return pl.pallas_call(
        _suffix_body,
        grid=(GRID,),
        in_specs=[
            pl.BlockSpec((R, 1), lambda i: (i, 0)),
            pl.BlockSpec((1, BINS), lambda i: (0, 0)),
        ],
        out_specs=pl.BlockSpec((1, BINS), lambda i: (0, 0)),
        out_shape=jax.ShapeDtypeStruct((1, BINS), jnp.float32),
        scratch_shapes=[pltpu.VMEM((1, BINS), jnp.float32)],
    )(logits_col, edges)


def _dest_body(v_ref, t_ref, dest_ref, tot_ref, carry):
    i = pl.program_id(0)

    @pl.when(i == 0)
    def _init():
        carry[...] = jnp.zeros((1, 1), jnp.float32)

    v = v_ref[...]
    mask = jnp.where(v >= t_ref[...], 1.0, 0.0)
    rr = jax.lax.broadcasted_iota(jnp.int32, (R, R), 0)
    cc = jax.lax.broadcasted_iota(jnp.int32, (R, R), 1)
    tri = jnp.where(rr >= cc, 1.0, 0.0)
    prefix = jnp.dot(tri, mask, preferred_element_type=jnp.float32)
    dest = carry[...] + prefix - 1.0
    dest_i = jnp.where(mask > 0.0, dest.astype(jnp.int32), -1)
    dest_ref[...] = dest_i
    carry_new = carry[...] + jnp.sum(mask, keepdims=True)
    carry[...] = carry_new

    @pl.when(i == GRID - 1)
    def _fin():
        tot_ref[...] = carry_new.astype(jnp.int32)


def _dests(logits_col, t_edge):
    return pl.pallas_call(
        _dest_body,
        grid=(GRID,),
        in_specs=[
            pl.BlockSpec((R, 1), lambda i: (i, 0)),
            pl.BlockSpec((1, 1), lambda i: (0, 0)),
        ],
        out_specs=[
            pl.BlockSpec((R, 1), lambda i: (i, 0)),
            pl.BlockSpec((1, 1), lambda i: (0, 0)),
        ],
        out_shape=[
            jax.ShapeDtypeStruct((NPAD, 1), jnp.int32),
            jax.ShapeDtypeStruct((1, 1), jnp.int32),
        ],
        scratch_shapes=[pltpu.VMEM((1, 1), jnp.float32)],
    )(logits_col, t_edge)


_CBLK = [128] * 24 + [64]


def _compact_body(logits_hbm, dests_hbm, cvals_hbm, cidx_hbm, *scr):
    chunk_v, gidx_v, dchunk_v = scr[0], scr[1], scr[2]
    dls = scr[3:3 + len(_CBLK)]
    sem = scr[-1]
    c = lax.axis_index("c")
    s = lax.axis_index("s")
    wid = s * 2 + c
    lanes = lax.iota(jnp.int32, 16)
    pltpu.sync_copy(logits_hbm.at[pl.ds(wid * CHUNK, CHUNK)], chunk_v)
    pltpu.sync_copy(dests_hbm.at[pl.ds(wid * CHUNK, CHUNK)], dchunk_v)

    for i in range(CHUNK // 16):
        j = i * 16 + lanes
        gidx_v[pl.ds(i * 16, 16)] = wid * CHUNK + j
        d = dchunk_v[pl.ds(i * 16, 16)]
        ok = (d >= 0) & (d < MPAD)
        dd = jnp.where(ok, d, -1)
        blk, off = i // 8, (i % 8) * 16
        dls[blk][pl.ds(off, 16)] = dd

    copies = []
    pos = 0
    for blk, blen in enumerate(_CBLK):
        idxs = plsc.Indices(dls[blk], ignored_value=-1)
        copies.append(pltpu.async_copy(
            chunk_v.at[pl.ds(pos, blen)], cvals_hbm.at[idxs], sem))
        copies.append(pltpu.async_copy(
            gidx_v.at[pl.ds(pos, blen)], cidx_hbm.at[idxs], sem))
        pos += blen
    for cp in copies:
        cp.wait()


@functools.cache
def _compact_kernel():
    return pl.kernel(
        _compact_body,
        out_type=[
            jax.ShapeDtypeStruct((MPAD + 64,), jnp.float32),
            jax.ShapeDtypeStruct((MPAD + 64,), jnp.int32),
        ],
        mesh=_mesh(),
        scratch_types=[
            pltpu.VMEM((CHUNK,), jnp.float32),
            pltpu.VMEM((CHUNK,), jnp.int32),
            pltpu.VMEM((CHUNK,), jnp.int32),
        ] + [pltpu.VMEM((blen,), jnp.int32) for blen in _CBLK] + [
            pltpu.SemaphoreType.DMA,
        ],
    )


JCH = 1024


def _rank_body(vi_ref, vj_ref, ii_ref, jj_ref, m_ref, s_ref, mt_ref,
               rank_ref, prob_ref):
    g = pl.program_id(0)
    vi = vi_ref[...]
    ii = ii_ref[...]
    vj = vj_ref[...]
    jj = jj_ref[...]
    mtot = mt_ref[...]
    ipos = g * JCH + jax.lax.broadcasted_iota(jnp.int32, (JCH, 1), 0)
    jpos_full = jax.lax.broadcasted_iota(jnp.int32, (1, MPAD), 1)

    acc = jnp.zeros((JCH, 1), jnp.float32)
    for cc in range(MPAD // JCH):
        vjc = vj[:, cc * JCH:(cc + 1) * JCH]
        jjc = jj[:, cc * JCH:(cc + 1) * JCH]
        jpc = jpos_full[:, cc * JCH:(cc + 1) * JCH]
        jvalid = jpc < mtot
        before = (vjc > vi) | ((vjc == vi) & (jjc < ii))
        cnt = jnp.where(before & jvalid, 1.0, 0.0)
        acc = acc + jnp.sum(cnt, axis=1, keepdims=True)

    ivalid = ipos < mtot
    rank = jnp.where(ivalid, acc.astype(jnp.int32), MPAD + ipos)
    rank_ref[...] = rank
    p = jnp.exp(vi - m_ref[...]) / s_ref[...]
    prob_ref[...] = jnp.where(ivalid, p, 0.0)


def _rank(vals_col, vals_row, idx_col, idx_row, m, s, mtot):
    return pl.pallas_call(
        _rank_body,
        grid=(MPAD // JCH,),
        in_specs=[
            pl.BlockSpec((JCH, 1), lambda g: (g, 0)),
            pl.BlockSpec((1, MPAD), lambda g: (0, 0)),
            pl.BlockSpec((JCH, 1), lambda g: (g, 0)),
            pl.BlockSpec((1, MPAD), lambda g: (0, 0)),
            pl.BlockSpec((1, 1), lambda g: (0, 0)),
            pl.BlockSpec((1, 1), lambda g: (0, 0)),
            pl.BlockSpec((1, 1), lambda g: (0, 0)),
        ],
        out_specs=[
            pl.BlockSpec((JCH, 1), lambda g: (g, 0)),
            pl.BlockSpec((JCH, 1), lambda g: (g, 0)),
        ],
        out_shape=[
            jax.ShapeDtypeStruct((MPAD, 1), jnp.int32),
            jax.ShapeDtypeStruct((MPAD, 1), jnp.float32),
        ],
    )(vals_col, vals_row, idx_col, idx_row, m, s, mtot)


OUTB = 10048
TRASH0 = K + 8


def _final_body(ranks_hbm, probs_hbm, cidx_hbm, loc_hbm,
                outbuf_hbm, partials_hbm,
                rk_v, pb_v, ci_v, sd0, sd1, sd2, cbuf_v, pt_v, sem):
    c = lax.axis_index("c")
    s = lax.axis_index("s")
    wid = s * 2 + c
    lanes = lax.iota(jnp.int32, 16)
    onesv = jnp.ones((16,), jnp.int32)
    base = wid * SLOTS
    pltpu.sync_copy(ranks_hbm.at[pl.ds(base, SLOTS)], rk_v)
    pltpu.sync_copy(probs_hbm.at[pl.ds(base, SLOTS)], pb_v)
    pltpu.sync_copy(cidx_hbm.at[pl.ds(base, SLOTS)], ci_v)

    sds = (sd0, sd0, sd0, sd0, sd0, sd0, sd0, sd0,
           sd1, sd1, sd1, sd1, sd1, sd1, sd1, sd1,
           sd2, sd2, sd2, sd2)
    offs = (0, 1, 2, 3, 4, 5, 6, 7, 0, 1, 2, 3, 4, 5, 6, 7, 0, 1, 2, 3)

    zf = jnp.zeros((16,), jnp.float32)

    def _fire3(mk_idx, launch):
        for i in range(SLOTS // 16):
            sds[i][pl.ds(offs[i] * 16, 16)] = mk_idx(i)
        copies = [launch(plsc.Indices(sd0, ignored_value=-1), 0, 128),
                  launch(plsc.Indices(sd1, ignored_value=-1), 128, 128),
                  launch(plsc.Indices(sd2, ignored_value=-1), 256, 64)]
        for cp in copies:
            cp.wait()

    def _rank_dest(i):
        r = rk_v[pl.ds(i * 16, 16)]
        return jnp.where(r < K, r, -1)
    _fire3(_rank_dest,
           lambda sd, pos, n: pltpu.async_copy(
               pb_v.at[pl.ds(pos, n)], outbuf_hbm.at[sd], sem))

    maccs = []
    haccs = []
    for c3 in range(3):
        for i in range(SLOTS // 16):
            cbuf_v[pl.ds(i * 16, 16)] = zf

        def _loc_idx(i, c3=c3):
            r = rk_v[pl.ds(i * 16, 16)]
            idxc = jnp.clip(ci_v[pl.ds(i * 16, 16)], 0, N - 1)
            return jnp.where(r < K, idxc * 3 + c3, -1)
        _fire3(_loc_idx,
               lambda sd, pos, n: pltpu.async_copy(
                   loc_hbm.at[sd], cbuf_v.at[pl.ds(pos, n)], sem))
        macc = jnp.zeros((16,), jnp.float32)
        for i in range(SLOTS // 16):
            macc = macc + cbuf_v[pl.ds(i * 16, 16)]
        maccs.append(macc)

        for i in range(SLOTS // 16):
            cbuf_v[pl.ds(i * 16, 16)] = zf

        def _hi_idx(i, c3=c3):
            r = rk_v[pl.ds(i * 16, 16)]
            idxc = jnp.clip(ci_v[pl.ds(i * 16, 16)], 0, N - 1)
            return jnp.where(r == 0, idxc * 3 + c3, -1)
        _fire3(_hi_idx,
               lambda sd, pos, n: pltpu.async_copy(
                   loc_hbm.at[sd], cbuf_v.at[pl.ds(pos, n)], sem))
        hacc = jnp.zeros((16,), jnp.float32)
        for i in range(SLOTS // 16):
            hacc = hacc + cbuf_v[pl.ds(i * 16, 16)]
        haccs.append(hacc)

    for c3 in range(3):
        pt_v[...] = maccs[c3]
        pltpu.sync_copy(pt_v, partials_hbm.at[pl.ds((wid * 6 + c3) * 16, 16)])
        pt_v[...] = haccs[c3]
        pltpu.sync_copy(
            pt_v, partials_hbm.at[pl.ds((wid * 6 + 3 + c3) * 16, 16)])


@functools.cache
def _final_kernel():
    return pl.kernel(
        _final_body,
        out_type=[
            jax.ShapeDtypeStruct((OUTB,), jnp.float32),
            jax.ShapeDtypeStruct((NW * 6 * 16,), jnp.float32),
        ],
        mesh=_mesh(),
        scratch_types=[
            pltpu.VMEM((SLOTS,), jnp.int32),
            pltpu.VMEM((SLOTS,), jnp.float32),
            pltpu.VMEM((SLOTS,), jnp.int32),
            pltpu.VMEM((128,), jnp.int32),
            pltpu.VMEM((128,), jnp.int32),
            pltpu.VMEM((64,), jnp.int32),
            pltpu.VMEM((SLOTS,), jnp.float32),
            pltpu.VMEM((16,), jnp.float32),
            pltpu.SemaphoreType.DMA,
        ],
    )


def kernel(point_features, point_locations, W1, b1, W2, b2):
    xp = jnp.pad(point_features, ((0, NPAD - N), (0, 0)))
    b1r = b1.reshape(1, HIDDEN)
    w2r = W2.reshape(1, HIDDEN)
    b2r = b2.reshape(1, 1)
    logits3, m, s, mn = _mlp_logits(xp, W1, b1r, w2r, b2r)
    logits = logits3.reshape(NPAD)
    logits_col = logits3.reshape(NPAD, 1)

    ar = jnp.arange(BINS, dtype=jnp.int32)
    arf = ar.astype(jnp.float32)
    width1 = (m[0, 0] - mn[0, 0]) / BINS
    edges1 = (mn[0, 0] + arf * width1).reshape(1, BINS)
    suf1 = _suffix_counts(logits_col, edges1)[0]
    b1i = jnp.max(jnp.where(suf1 >= KSEL, ar, 0))
    t1 = edges1[0, b1i]

    width2 = width1 / BINS
    edges2 = (t1 + arf * width2).reshape(1, BINS)
    suf2 = _suffix_counts(logits_col, edges2)[0]
    b2i = jnp.max(jnp.where(suf2 >= KSEL, ar, 0))
    t_edge = edges2[0, b2i].reshape(1, 1)

    dests, mtot = _dests(logits_col, t_edge)
    mtot = jnp.minimum(mtot, MPAD)

    cvals, cidx = _compact_kernel()(logits, dests.reshape(NPAD))
    vals_m = cvals[:MPAD]
    idx_m = cidx[:MPAD]
    ranks, probs = _rank(vals_m.reshape(MPAD, 1), vals_m.reshape(1, MPAD),
                         idx_m.reshape(MPAD, 1), idx_m.reshape(1, MPAD),
                         m, s, mtot)

    locflat = jnp.pad(point_locations, ((0, 8), (0, 0))).reshape(-1)
    outbuf, partials = _final_kernel()(ranks.reshape(MPAD),
                                       probs.reshape(MPAD), idx_m, locflat)
    pmat = partials.reshape(NW, 6, 16).sum(axis=(0, 2))
    mean_location = pmat[0:3] / K
    highest = pmat[3:6]
    return jnp.concatenate([outbuf[:K], mean_location, highest], axis=0)


# speedup vs baseline: 18.8490x; 1.0683x over previous
import functools

import jax
import jax.numpy as jnp
from jax import lax
from jax.experimental import pallas as pl
from jax.experimental.pallas import tpu as pltpu
from jax.experimental.pallas import tpu_sc as plsc

N = 100000
IN_DIM = 256
HIDDEN = 512
K = 10000
R = 1024
GRID = 98
NPAD = R * GRID
NW = 32
CHUNK = NPAD // NW
BINS = 512
MPAD = 10240
SLOTS = MPAD // NW
KSEL = K + 8
NEG_INF = float("-inf")
POS_INF = float("inf")


@functools.cache
def _mesh():
    return plsc.VectorSubcoreMesh(core_axis_name="c", subcore_axis_name="s",
                                  num_cores=2, num_subcores=16)


def _mlp_body(x_ref, w1_ref, b1_ref, w2_ref, b2_ref,
              logits_ref, m_ref, s_ref, mn_ref, m_acc, s_acc, mn_acc):
    i = pl.program_id(0)
    x = x_ref[...]
    h = jnp.dot(x, w1_ref[...], preferred_element_type=jnp.float32)
    h = jnp.maximum(h + b1_ref[...], 0.0)
    hw = h * w2_ref[...]
    logit = jnp.sum(jnp.reshape(hw, (8, 128, HIDDEN)), axis=-1) + b2_ref[...]
    row = (i * R
           + jax.lax.broadcasted_iota(jnp.int32, (8, 128), 0) * 128
           + jax.lax.broadcasted_iota(jnp.int32, (8, 128), 1))
    valid = row < N
    logit = jnp.where(valid, logit, NEG_INF)
    logits_ref[0] = logit

    @pl.when(i == 0)
    def _init():
        m_acc[...] = jnp.full((8, 128), NEG_INF, jnp.float32)
        s_acc[...] = jnp.zeros((8, 128), jnp.float32)
        mn_acc[...] = jnp.full((8, 128), POS_INF, jnp.float32)

    m_old = m_acc[...]
    m_new = jnp.maximum(m_old, logit)
    scale = jnp.where(m_old == NEG_INF, 0.0, jnp.exp(m_old - m_new))
    term = jnp.where(logit == NEG_INF, 0.0, jnp.exp(logit - m_new))
    s_new = s_acc[...] * scale + term
    m_acc[...] = m_new
    s_acc[...] = s_new
    mn_new = jnp.minimum(mn_acc[...], jnp.where(valid, logit, POS_INF))
    mn_acc[...] = mn_new

    @pl.when(i == GRID - 1)
    def _fin():
        m_all = jnp.max(m_new)
        s_all = jnp.sum(s_new * jnp.exp(m_new - m_all))
        m_ref[...] = jnp.reshape(m_all, (1, 1))
        s_ref[...] = jnp.reshape(s_all, (1, 1))
        mn_ref[...] = jnp.reshape(jnp.min(mn_new), (1, 1))


def _mlp_logits(xp, W1, b1r, w2r, b2r):
    return pl.pallas_call(
        _mlp_body,
        grid=(GRID,),
        in_specs=[
            pl.BlockSpec((R, IN_DIM), lambda i: (i, 0)),
            pl.BlockSpec((IN_DIM, HIDDEN), lambda i: (0, 0)),
            pl.BlockSpec((1, HIDDEN), lambda i: (0, 0)),
            pl.BlockSpec((1, HIDDEN), lambda i: (0, 0)),
            pl.BlockSpec((1, 1), lambda i: (0, 0)),
        ],
        out_specs=[
            pl.BlockSpec((1, 8, 128), lambda i: (i, 0, 0)),
            pl.BlockSpec((1, 1), lambda i: (0, 0)),
            pl.BlockSpec((1, 1), lambda i: (0, 0)),
            pl.BlockSpec((1, 1), lambda i: (0, 0)),
        ],
        out_shape=[
            jax.ShapeDtypeStruct((GRID, 8, 128), jnp.float32),
            jax.ShapeDtypeStruct((1, 1), jnp.float32),
            jax.ShapeDtypeStruct((1, 1), jnp.float32),
            jax.ShapeDtypeStruct((1, 1), jnp.float32),
        ],
        scratch_shapes=[
            pltpu.VMEM((8, 128), jnp.float32),
            pltpu.VMEM((8, 128), jnp.float32),
            pltpu.VMEM((8, 128), jnp.float32),
        ],
    )(xp, W1, b1r, w2r, b2r)


def _suffix_body(v_ref, edges_ref, out_ref, acc):
    i = pl.program_id(0)

    @pl.when(i == 0)
    def _init():
        acc[...] = jnp.zeros((1, BINS), jnp.float32)

    v = v_ref[...]
    ge = jnp.where(v >= edges_ref[...], 1.0, 0.0)
    acc[...] += jnp.sum(ge, axis=0, keepdims=True)

    @pl.when(i == GRID - 1)
    def _fin():
        out_ref[...] = acc[...]


def _suffix_counts(logits_col, edges):
    return pl.pallas_call(
        _suffix_body,
        grid=(GRID,),
        in_specs=[
            pl.BlockSpec((R, 1), lambda i: (i, 0)),
            pl.BlockSpec((1, BINS), lambda i: (0, 0)),
        ],
        out_specs=pl.BlockSpec((1, BINS), lambda i: (0, 0)),
        out_shape=jax.ShapeDtypeStruct((1, BINS), jnp.float32),
        scratch_shapes=[pltpu.VMEM((1, BINS), jnp.float32)],
    )(logits_col, edges)


def _dest_body(v_ref, t_ref, dest_ref, tot_ref, carry):
    i = pl.program_id(0)

    @pl.when(i == 0)
    def _init():
        carry[...] = jnp.zeros((1, 1), jnp.float32)

    v = v_ref[...]
    mask = jnp.where(v >= t_ref[...], 1.0, 0.0)
    rr = jax.lax.broadcasted_iota(jnp.int32, (R, R), 0)
    cc = jax.lax.broadcasted_iota(jnp.int32, (R, R), 1)
    tri = jnp.where(rr >= cc, 1.0, 0.0)
    prefix = jnp.dot(tri, mask, preferred_element_type=jnp.float32)
    dest = carry[...] + prefix - 1.0
    dest_i = jnp.where(mask > 0.0, dest.astype(jnp.int32), -1)
    dest_ref[...] = dest_i
    carry_new = carry[...] + jnp.sum(mask, keepdims=True)
    carry[...] = carry_new

    @pl.when(i == GRID - 1)
    def _fin():
        tot_ref[...] = carry_new.astype(jnp.int32)


def _dests(logits_col, t_edge):
    return pl.pallas_call(
        _dest_body,
        grid=(GRID,),
        in_specs=[
            pl.BlockSpec((R, 1), lambda i: (i, 0)),
            pl.BlockSpec((1, 1), lambda i: (0, 0)),
        ],
        out_specs=[
            pl.BlockSpec((R, 1), lambda i: (i, 0)),
            pl.BlockSpec((1, 1), lambda i: (0, 0)),
        ],
        out_shape=[
            jax.ShapeDtypeStruct((NPAD, 1), jnp.int32),
            jax.ShapeDtypeStruct((1, 1), jnp.int32),
        ],
        scratch_shapes=[pltpu.VMEM((1, 1), jnp.float32)],
    )(logits_col, t_edge)


_CBLK = [128] * 24 + [64]


def _compact_body(logits_hbm, dests_hbm, cvals_hbm, cidx_hbm, *scr):
    chunk_v, gidx_v, dchunk_v = scr[0], scr[1], scr[2]
    dls = scr[3:3 + len(_CBLK)]
    sem = scr[-1]
    c = lax.axis_index("c")
    s = lax.axis_index("s")
    wid = s * 2 + c
    lanes = lax.iota(jnp.int32, 16)
    pltpu.sync_copy(logits_hbm.at[pl.ds(wid * CHUNK, CHUNK)], chunk_v)
    pltpu.sync_copy(dests_hbm.at[pl.ds(wid * CHUNK, CHUNK)], dchunk_v)

    for i in range(CHUNK // 16):
        j = i * 16 + lanes
        gidx_v[pl.ds(i * 16, 16)] = wid * CHUNK + j
        d = dchunk_v[pl.ds(i * 16, 16)]
        ok = (d >= 0) & (d < MPAD)
        dd = jnp.where(ok, d, -1)
        blk, off = i // 8, (i % 8) * 16
        dls[blk][pl.ds(off, 16)] = dd

    copies = []
    pos = 0
    for blk, blen in enumerate(_CBLK):
        idxs = plsc.Indices(dls[blk], ignored_value=-1)
        copies.append(pltpu.async_copy(
            chunk_v.at[pl.ds(pos, blen)], cvals_hbm.at[idxs], sem))
        copies.append(pltpu.async_copy(
            gidx_v.at[pl.ds(pos, blen)], cidx_hbm.at[idxs], sem))
        pos += blen
    for cp in copies:
        cp.wait()


@functools.cache
def _compact_kernel():
    return pl.kernel(
        _compact_body,
        out_type=[
            jax.ShapeDtypeStruct((MPAD + 64,), jnp.float32),
            jax.ShapeDtypeStruct((MPAD + 64,), jnp.int32),
        ],
        mesh=_mesh(),
        scratch_types=[
            pltpu.VMEM((CHUNK,), jnp.float32),
            pltpu.VMEM((CHUNK,), jnp.int32),
            pltpu.VMEM((CHUNK,), jnp.int32),
        ] + [pltpu.VMEM((blen,), jnp.int32) for blen in _CBLK] + [
            pltpu.SemaphoreType.DMA,
        ],
    )


JCH = 1024


def _rank_body(vi_ref, vj_ref, ii_ref, jj_ref, m_ref, s_ref,
               rank_ref, prob_ref):
    vi = vi_ref[...]
    ii = ii_ref[...]
    vj = vj_ref[...]
    jj = jj_ref[...]

    acc = jnp.zeros((JCH, 1), jnp.float32)
    for cc in range(MPAD // JCH):
        vjc = vj[:, cc * JCH:(cc + 1) * JCH]
        jjc = jj[:, cc * JCH:(cc + 1) * JCH]
        before = (vjc > vi) | ((vjc == vi) & (jjc < ii))
        acc = acc + jnp.sum(jnp.where(before, 1.0, 0.0), axis=1,
                            keepdims=True)

    rank_ref[...] = acc.astype(jnp.int32)
    prob_ref[...] = jnp.exp(vi - m_ref[...]) / s_ref[...]


def _rank(vals_col, vals_row, idx_col, idx_row, m, s):
    return pl.pallas_call(
        _rank_body,
        grid=(MPAD // JCH,),
        in_specs=[
            pl.BlockSpec((JCH, 1), lambda g: (g, 0)),
            pl.BlockSpec((1, MPAD), lambda g: (0, 0)),
            pl.BlockSpec((JCH, 1), lambda g: (g, 0)),
            pl.BlockSpec((1, MPAD), lambda g: (0, 0)),
            pl.BlockSpec((1, 1), lambda g: (0, 0)),
            pl.BlockSpec((1, 1), lambda g: (0, 0)),
        ],
        out_specs=[
            pl.BlockSpec((JCH, 1), lambda g: (g, 0)),
            pl.BlockSpec((JCH, 1), lambda g: (g, 0)),
        ],
        out_shape=[
            jax.ShapeDtypeStruct((MPAD, 1), jnp.int32),
            jax.ShapeDtypeStruct((MPAD, 1), jnp.float32),
        ],
    )(vals_col, vals_row, idx_col, idx_row, m, s)


OUTB = 10048
TRASH0 = K + 8


def _final_body(ranks_hbm, probs_hbm, cidx_hbm, loc_hbm,
                outbuf_hbm, partials_hbm,
                rk_v, pb_v, ci_v, sd0, sd1, sd2, cbuf_v, pt_v, sem):
    c = lax.axis_index("c")
    s = lax.axis_index("s")
    wid = s * 2 + c
    lanes = lax.iota(jnp.int32, 16)
    onesv = jnp.ones((16,), jnp.int32)
    base = wid * SLOTS
    pltpu.sync_copy(ranks_hbm.at[pl.ds(base, SLOTS)], rk_v)
    pltpu.sync_copy(probs_hbm.at[pl.ds(base, SLOTS)], pb_v)
    pltpu.sync_copy(cidx_hbm.at[pl.ds(base, SLOTS)], ci_v)

    sds = (sd0, sd0, sd0, sd0, sd0, sd0, sd0, sd0,
           sd1, sd1, sd1, sd1, sd1, sd1, sd1, sd1,
           sd2, sd2, sd2, sd2)
    offs = (0, 1, 2, 3, 4, 5, 6, 7, 0, 1, 2, 3, 4, 5, 6, 7, 0, 1, 2, 3)

    zf = jnp.zeros((16,), jnp.float32)

    def _fire3(mk_idx, launch):
        for i in range(SLOTS // 16):
            sds[i][pl.ds(offs[i] * 16, 16)] = mk_idx(i)
        copies = [launch(plsc.Indices(sd0, ignored_value=-1), 0, 128),
                  launch(plsc.Indices(sd1, ignored_value=-1), 128, 128),
                  launch(plsc.Indices(sd2, ignored_value=-1), 256, 64)]
        for cp in copies:
            cp.wait()

    def _rank_dest(i):
        r = rk_v[pl.ds(i * 16, 16)]
        return jnp.where(r < K, r, -1)
    _fire3(_rank_dest,
           lambda sd, pos, n: pltpu.async_copy(
               pb_v.at[pl.ds(pos, n)], outbuf_hbm.at[sd], sem))

    maccs = []
    haccs = []
    for c3 in range(3):
        for i in range(SLOTS // 16):
            cbuf_v[pl.ds(i * 16, 16)] = zf

        def _loc_idx(i, c3=c3):
            r = rk_v[pl.ds(i * 16, 16)]
            idxc = jnp.clip(ci_v[pl.ds(i * 16, 16)], 0, N - 1)
            return jnp.where(r < K, idxc * 3 + c3, -1)
        _fire3(_loc_idx,
               lambda sd, pos, n: pltpu.async_copy(
                   loc_hbm.at[sd], cbuf_v.at[pl.ds(pos, n)], sem))
        macc = jnp.zeros((16,), jnp.float32)
        for i in range(SLOTS // 16):
            macc = macc + cbuf_v[pl.ds(i * 16, 16)]
        maccs.append(macc)

        for i in range(SLOTS // 16):
            cbuf_v[pl.ds(i * 16, 16)] = zf

        def _hi_idx(i, c3=c3):
            r = rk_v[pl.ds(i * 16, 16)]
            idxc = jnp.clip(ci_v[pl.ds(i * 16, 16)], 0, N - 1)
            return jnp.where(r == 0, idxc * 3 + c3, -1)
        _fire3(_hi_idx,
               lambda sd, pos, n: pltpu.async_copy(
                   loc_hbm.at[sd], cbuf_v.at[pl.ds(pos, n)], sem))
        hacc = jnp.zeros((16,), jnp.float32)
        for i in range(SLOTS // 16):
            hacc = hacc + cbuf_v[pl.ds(i * 16, 16)]
        haccs.append(hacc)

    for c3 in range(3):
        pt_v[...] = maccs[c3]
        pltpu.sync_copy(pt_v, partials_hbm.at[pl.ds((wid * 6 + c3) * 16, 16)])
        pt_v[...] = haccs[c3]
        pltpu.sync_copy(
            pt_v, partials_hbm.at[pl.ds((wid * 6 + 3 + c3) * 16, 16)])


@functools.cache
def _final_kernel():
    return pl.kernel(
        _final_body,
        out_type=[
            jax.ShapeDtypeStruct((OUTB,), jnp.float32),
            jax.ShapeDtypeStruct((NW * 6 * 16,), jnp.float32),
        ],
        mesh=_mesh(),
        scratch_types=[
            pltpu.VMEM((SLOTS,), jnp.int32),
            pltpu.VMEM((SLOTS,), jnp.float32),
            pltpu.VMEM((SLOTS,), jnp.int32),
            pltpu.VMEM((128,), jnp.int32),
            pltpu.VMEM((128,), jnp.int32),
            pltpu.VMEM((64,), jnp.int32),
            pltpu.VMEM((SLOTS,), jnp.float32),
            pltpu.VMEM((16,), jnp.float32),
            pltpu.SemaphoreType.DMA,
        ],
    )


def kernel(point_features, point_locations, W1, b1, W2, b2):
    xp = jnp.pad(point_features, ((0, NPAD - N), (0, 0)))
    b1r = b1.reshape(1, HIDDEN)
    w2r = W2.reshape(1, HIDDEN)
    b2r = b2.reshape(1, 1)
    logits3, m, s, mn = _mlp_logits(xp, W1, b1r, w2r, b2r)
    logits = logits3.reshape(NPAD)
    logits_col = logits3.reshape(NPAD, 1)

    ar = jnp.arange(BINS, dtype=jnp.int32)
    arf = ar.astype(jnp.float32)
    width1 = (m[0, 0] - mn[0, 0]) / BINS
    edges1 = (mn[0, 0] + arf * width1).reshape(1, BINS)
    suf1 = _suffix_counts(logits_col, edges1)[0]
    b1i = jnp.max(jnp.where(suf1 >= KSEL, ar, 0))
    t1 = edges1[0, b1i]

    width2 = width1 / BINS
    edges2 = (t1 + arf * width2).reshape(1, BINS)
    suf2 = _suffix_counts(logits_col, edges2)[0]
    b2i = jnp.max(jnp.where(suf2 >= KSEL, ar, 0))
    t_edge = edges2[0, b2i].reshape(1, 1)

    dests, mtot = _dests(logits_col, t_edge)
    mtot = jnp.minimum(mtot, MPAD)

    cvals, cidx = _compact_kernel()(logits, dests.reshape(NPAD))
    ar10 = jnp.arange(MPAD, dtype=jnp.int32)
    valid = ar10 < mtot[0, 0]
    vals_m = jnp.where(valid, cvals[:MPAD], NEG_INF)
    idx_m = jnp.where(valid, cidx[:MPAD], NPAD + ar10)
    ranks, probs = _rank(vals_m.reshape(MPAD, 1), vals_m.reshape(1, MPAD),
                         idx_m.reshape(MPAD, 1), idx_m.reshape(1, MPAD),
                         m, s)

    locflat = jnp.pad(point_locations, ((0, 8), (0, 0))).reshape(-1)
    outbuf, partials = _final_kernel()(ranks.reshape(MPAD),
                                       probs.reshape(MPAD), idx_m, locflat)
    pmat = partials.reshape(NW, 6, 16).sum(axis=(0, 2))
    mean_location = pmat[0:3] / K
    highest = pmat[3:6]
    return jnp.concatenate([outbuf[:K], mean_location, highest], axis=0)


# speedup vs baseline: 21.4170x; 1.1362x over previous
import functools

import jax
import jax.numpy as jnp
from jax import lax
from jax.experimental import pallas as pl
from jax.experimental.pallas import tpu as pltpu
from jax.experimental.pallas import tpu_sc as plsc

N = 100000
IN_DIM = 256
HIDDEN = 512
K = 10000
R = 1024
GRID = 98
NPAD = R * GRID
NW = 32
CHUNK = NPAD // NW
BINS = 512
MPAD = 10240
SLOTS = MPAD // NW
KSEL = K + 8
NEG_INF = float("-inf")
POS_INF = float("inf")


@functools.cache
def _mesh():
    return plsc.VectorSubcoreMesh(core_axis_name="c", subcore_axis_name="s",
                                  num_cores=2, num_subcores=16)


def _mlp_body(x_ref, w1_ref, b1_ref, w2_ref, b2_ref,
              logits_ref, m_ref, s_ref, mn_ref, m_acc, s_acc, mn_acc):
    i = pl.program_id(0)
    x = x_ref[...]
    h = jnp.dot(x, w1_ref[...], preferred_element_type=jnp.float32)
    h = jnp.maximum(h + b1_ref[...], 0.0)
    hw = h * w2_ref[...]
    logit = jnp.sum(jnp.reshape(hw, (8, 128, HIDDEN)), axis=-1) + b2_ref[...]
    row = (i * R
           + jax.lax.broadcasted_iota(jnp.int32, (8, 128), 0) * 128
           + jax.lax.broadcasted_iota(jnp.int32, (8, 128), 1))
    valid = row < N
    logit = jnp.where(valid, logit, NEG_INF)
    logits_ref[0] = logit

    @pl.when(i == 0)
    def _init():
        m_acc[...] = jnp.full((8, 128), NEG_INF, jnp.float32)
        s_acc[...] = jnp.zeros((8, 128), jnp.float32)
        mn_acc[...] = jnp.full((8, 128), POS_INF, jnp.float32)

    m_old = m_acc[...]
    m_new = jnp.maximum(m_old, logit)
    scale = jnp.where(m_old == NEG_INF, 0.0, jnp.exp(m_old - m_new))
    term = jnp.where(logit == NEG_INF, 0.0, jnp.exp(logit - m_new))
    s_new = s_acc[...] * scale + term
    m_acc[...] = m_new
    s_acc[...] = s_new
    mn_new = jnp.minimum(mn_acc[...], jnp.where(valid, logit, POS_INF))
    mn_acc[...] = mn_new

    @pl.when(i == GRID - 1)
    def _fin():
        m_all = jnp.max(m_new)
        s_all = jnp.sum(s_new * jnp.exp(m_new - m_all))
        m_ref[...] = jnp.reshape(m_all, (1, 1))
        s_ref[...] = jnp.reshape(s_all, (1, 1))
        mn_ref[...] = jnp.reshape(jnp.min(mn_new), (1, 1))


def _mlp_logits(xp, W1, b1r, w2r, b2r):
    return pl.pallas_call(
        _mlp_body,
        grid=(GRID,),
        in_specs=[
            pl.BlockSpec((R, IN_DIM), lambda i: (i, 0)),
            pl.BlockSpec((IN_DIM, HIDDEN), lambda i: (0, 0)),
            pl.BlockSpec((1, HIDDEN), lambda i: (0, 0)),
            pl.BlockSpec((1, HIDDEN), lambda i: (0, 0)),
            pl.BlockSpec((1, 1), lambda i: (0, 0)),
        ],
        out_specs=[
            pl.BlockSpec((1, 8, 128), lambda i: (i, 0, 0)),
            pl.BlockSpec((1, 1), lambda i: (0, 0)),
            pl.BlockSpec((1, 1), lambda i: (0, 0)),
            pl.BlockSpec((1, 1), lambda i: (0, 0)),
        ],
        out_shape=[
            jax.ShapeDtypeStruct((GRID, 8, 128), jnp.float32),
            jax.ShapeDtypeStruct((1, 1), jnp.float32),
            jax.ShapeDtypeStruct((1, 1), jnp.float32),
            jax.ShapeDtypeStruct((1, 1), jnp.float32),
        ],
        scratch_shapes=[
            pltpu.VMEM((8, 128), jnp.float32),
            pltpu.VMEM((8, 128), jnp.float32),
            pltpu.VMEM((8, 128), jnp.float32),
        ],
    )(xp, W1, b1r, w2r, b2r)


def _suffix_body(v_ref, edges_ref, out_ref, acc):
    i = pl.program_id(0)

    @pl.when(i == 0)
    def _init():
        acc[...] = jnp.zeros((1, BINS), jnp.float32)

    v = v_ref[...]
    ge = jnp.where(v >= edges_ref[...], 1.0, 0.0)
    acc[...] += jnp.sum(ge, axis=0, keepdims=True)

    @pl.when(i == GRID - 1)
    def _fin():
        out_ref[...] = acc[...]


def _suffix_counts(logits_col, edges):
    return pl.pallas_call(
        _suffix_body,
        grid=(GRID,),
        in_specs=[
            pl.BlockSpec((R, 1), lambda i: (i, 0)),
            pl.BlockSpec((1, BINS), lambda i: (0, 0)),
        ],
        out_specs=pl.BlockSpec((1, BINS), lambda i: (0, 0)),
        out_shape=jax.ShapeDtypeStruct((1, BINS), jnp.float32),
        scratch_shapes=[pltpu.VMEM((1, BINS), jnp.float32)],
    )(logits_col, edges)


def _dest_body(v_ref, t_ref, dest_ref, tot_ref, carry):
    i = pl.program_id(0)

    @pl.when(i == 0)
    def _init():
        carry[...] = jnp.zeros((1, 1), jnp.float32)

    v = v_ref[...]
    mask = jnp.where(v >= t_ref[...], 1.0, 0.0)
    rr = jax.lax.broadcasted_iota(jnp.int32, (R, R), 0)
    cc = jax.lax.broadcasted_iota(jnp.int32, (R, R), 1)
    tri = jnp.where(rr >= cc, 1.0, 0.0).astype(jnp.bfloat16)
    prefix = jnp.dot(tri, mask.astype(jnp.bfloat16),
                     preferred_element_type=jnp.float32)
    dest = carry[...] + prefix - 1.0
    di = dest.astype(jnp.int32)
    pi = (di & 63) * 160 + (di >> 6)
    dest_i = jnp.where((mask > 0.0) & (di < MPAD), pi, -1)
    dest_ref[...] = dest_i
    carry_new = carry[...] + jnp.sum(mask, keepdims=True)
    carry[...] = carry_new

    @pl.when(i == GRID - 1)
    def _fin():
        tot_ref[...] = carry_new.astype(jnp.int32)


def _dests(logits_col, t_edge):
    return pl.pallas_call(
        _dest_body,
        grid=(GRID,),
        in_specs=[
            pl.BlockSpec((R, 1), lambda i: (i, 0)),
            pl.BlockSpec((1, 1), lambda i: (0, 0)),
        ],
        out_specs=[
            pl.BlockSpec((R, 1), lambda i: (i, 0)),
            pl.BlockSpec((1, 1), lambda i: (0, 0)),
        ],
        out_shape=[
            jax.ShapeDtypeStruct((NPAD, 1), jnp.int32),
            jax.ShapeDtypeStruct((1, 1), jnp.int32),
        ],
        scratch_shapes=[pltpu.VMEM((1, 1), jnp.float32)],
    )(logits_col, t_edge)


_CBLK = [128] * 24 + [64]


def _compact_body(logits_hbm, dests_hbm, cvals_hbm, cidx_hbm, *scr):
    chunk_v, gidx_v, dchunk_v = scr[0], scr[1], scr[2]
    dls = scr[3:3 + len(_CBLK)]
    sem = scr[-1]
    c = lax.axis_index("c")
    s = lax.axis_index("s")
    wid = s * 2 + c
    lanes = lax.iota(jnp.int32, 16)
    pltpu.sync_copy(logits_hbm.at[pl.ds(wid * CHUNK, CHUNK)], chunk_v)
    pltpu.sync_copy(dests_hbm.at[pl.ds(wid * CHUNK, CHUNK)], dchunk_v)

    for i in range(CHUNK // 16):
        j = i * 16 + lanes
        gidx_v[pl.ds(i * 16, 16)] = wid * CHUNK + j
        blk, off = i // 8, (i % 8) * 16
        dls[blk][pl.ds(off, 16)] = dchunk_v[pl.ds(i * 16, 16)]

    copies = []
    pos = 0
    for blk, blen in enumerate(_CBLK):
        idxs = plsc.Indices(dls[blk], ignored_value=-1)
        copies.append(pltpu.async_copy(
            chunk_v.at[pl.ds(pos, blen)], cvals_hbm.at[idxs], sem))
        copies.append(pltpu.async_copy(
            gidx_v.at[pl.ds(pos, blen)], cidx_hbm.at[idxs], sem))
        pos += blen
    for cp in copies:
        cp.wait()


@functools.cache
def _compact_kernel():
    return pl.kernel(
        _compact_body,
        out_type=[
            jax.ShapeDtypeStruct((MPAD + 64,), jnp.float32),
            jax.ShapeDtypeStruct((MPAD + 64,), jnp.int32),
        ],
        mesh=_mesh(),
        scratch_types=[
            pltpu.VMEM((CHUNK,), jnp.float32),
            pltpu.VMEM((CHUNK,), jnp.int32),
            pltpu.VMEM((CHUNK,), jnp.int32),
        ] + [pltpu.VMEM((blen,), jnp.int32) for blen in _CBLK] + [
            pltpu.SemaphoreType.DMA,
        ],
    )


JCH = 1024


def _rank_body(vi_ref, vj_ref, ii_ref, jj_ref, m_ref, s_ref,
               rank_ref, prob_ref):
    vi = vi_ref[...]
    ii = ii_ref[...]
    vj = vj_ref[...]
    jj = jj_ref[...]

    acc = jnp.zeros((JCH, 1), jnp.float32)
    for cc in range(MPAD // JCH):
        vjc = vj[:, cc * JCH:(cc + 1) * JCH]
        jjc = jj[:, cc * JCH:(cc + 1) * JCH]
        before = (vjc > vi) | ((vjc == vi) & (jjc < ii))
        acc = acc + jnp.sum(jnp.where(before, 1.0, 0.0), axis=1,
                            keepdims=True)

    rank_ref[...] = acc.astype(jnp.int32)
    prob_ref[...] = jnp.exp(vi - m_ref[...]) / s_ref[...]


def _rank(vals_col, vals_row, idx_col, idx_row, m, s):
    return pl.pallas_call(
        _rank_body,
        grid=(MPAD // JCH,),
        in_specs=[
            pl.BlockSpec((JCH, 1), lambda g: (g, 0)),
            pl.BlockSpec((1, MPAD), lambda g: (0, 0)),
            pl.BlockSpec((JCH, 1), lambda g: (g, 0)),
            pl.BlockSpec((1, MPAD), lambda g: (0, 0)),
            pl.BlockSpec((1, 1), lambda g: (0, 0)),
            pl.BlockSpec((1, 1), lambda g: (0, 0)),
        ],
        out_specs=[
            pl.BlockSpec((JCH, 1), lambda g: (g, 0)),
            pl.BlockSpec((JCH, 1), lambda g: (g, 0)),
        ],
        out_shape=[
            jax.ShapeDtypeStruct((MPAD, 1), jnp.int32),
            jax.ShapeDtypeStruct((MPAD, 1), jnp.float32),
        ],
    )(vals_col, vals_row, idx_col, idx_row, m, s)


OUTB = 10048
TRASH0 = K + 8


def _final_body(ranks_hbm, probs_hbm, cidx_hbm, loc_hbm,
                outbuf_hbm, partials_hbm,
                rk_v, pb_v, ci_v, sd0, sd1, sd2, cbuf_v, pt_v, sem):
    c = lax.axis_index("c")
    s = lax.axis_index("s")
    wid = s * 2 + c
    lanes = lax.iota(jnp.int32, 16)
    onesv = jnp.ones((16,), jnp.int32)
    base = wid * SLOTS
    pltpu.sync_copy(ranks_hbm.at[pl.ds(base, SLOTS)], rk_v)
    pltpu.sync_copy(probs_hbm.at[pl.ds(base, SLOTS)], pb_v)
    pltpu.sync_copy(cidx_hbm.at[pl.ds(base, SLOTS)], ci_v)

    sds = (sd0, sd0, sd0, sd0, sd0, sd0, sd0, sd0,
           sd1, sd1, sd1, sd1, sd1, sd1, sd1, sd1,
           sd2, sd2, sd2, sd2)
    offs = (0, 1, 2, 3, 4, 5, 6, 7, 0, 1, 2, 3, 4, 5, 6, 7, 0, 1, 2, 3)

    zf = jnp.zeros((16,), jnp.float32)

    def _fire3(mk_idx, launch):
        for i in range(SLOTS // 16):
            sds[i][pl.ds(offs[i] * 16, 16)] = mk_idx(i)
        copies = [launch(plsc.Indices(sd0, ignored_value=-1), 0, 128),
                  launch(plsc.Indices(sd1, ignored_value=-1), 128, 128),
                  launch(plsc.Indices(sd2, ignored_value=-1), 256, 64)]
        for cp in copies:
            cp.wait()

    def _rank_dest(i):
        r = rk_v[pl.ds(i * 16, 16)]
        return jnp.where(r < K, r, -1)
    _fire3(_rank_dest,
           lambda sd, pos, n: pltpu.async_copy(
               pb_v.at[pl.ds(pos, n)], outbuf_hbm.at[sd], sem))

    maccs = []
    haccs = []
    for c3 in range(3):
        for i in range(SLOTS // 16):
            cbuf_v[pl.ds(i * 16, 16)] = zf

        def _loc_idx(i, c3=c3):
            r = rk_v[pl.ds(i * 16, 16)]
            idxc = jnp.clip(ci_v[pl.ds(i * 16, 16)], 0, N - 1)
            return jnp.where(r < K, idxc * 3 + c3, -1)
        _fire3(_loc_idx,
               lambda sd, pos, n: pltpu.async_copy(
                   loc_hbm.at[sd], cbuf_v.at[pl.ds(pos, n)], sem))
        macc = jnp.zeros((16,), jnp.float32)
        for i in range(SLOTS // 16):
            macc = macc + cbuf_v[pl.ds(i * 16, 16)]
        maccs.append(macc)

        for i in range(SLOTS // 16):
            cbuf_v[pl.ds(i * 16, 16)] = zf

        def _hi_idx(i, c3=c3):
            r = rk_v[pl.ds(i * 16, 16)]
            idxc = jnp.clip(ci_v[pl.ds(i * 16, 16)], 0, N - 1)
            return jnp.where(r == 0, idxc * 3 + c3, -1)
        _fire3(_hi_idx,
               lambda sd, pos, n: pltpu.async_copy(
                   loc_hbm.at[sd], cbuf_v.at[pl.ds(pos, n)], sem))
        hacc = jnp.zeros((16,), jnp.float32)
        for i in range(SLOTS // 16):
            hacc = hacc + cbuf_v[pl.ds(i * 16, 16)]
        haccs.append(hacc)

    for c3 in range(3):
        pt_v[...] = maccs[c3]
        pltpu.sync_copy(pt_v, partials_hbm.at[pl.ds((wid * 6 + c3) * 16, 16)])
        pt_v[...] = haccs[c3]
        pltpu.sync_copy(
            pt_v, partials_hbm.at[pl.ds((wid * 6 + 3 + c3) * 16, 16)])


@functools.cache
def _final_kernel():
    return pl.kernel(
        _final_body,
        out_type=[
            jax.ShapeDtypeStruct((OUTB,), jnp.float32),
            jax.ShapeDtypeStruct((NW * 6 * 16,), jnp.float32),
        ],
        mesh=_mesh(),
        scratch_types=[
            pltpu.VMEM((SLOTS,), jnp.int32),
            pltpu.VMEM((SLOTS,), jnp.float32),
            pltpu.VMEM((SLOTS,), jnp.int32),
            pltpu.VMEM((128,), jnp.int32),
            pltpu.VMEM((128,), jnp.int32),
            pltpu.VMEM((64,), jnp.int32),
            pltpu.VMEM((SLOTS,), jnp.float32),
            pltpu.VMEM((16,), jnp.float32),
            pltpu.SemaphoreType.DMA,
        ],
    )


def kernel(point_features, point_locations, W1, b1, W2, b2):
    xp = point_features
    b1r = b1.reshape(1, HIDDEN)
    w2r = W2.reshape(1, HIDDEN)
    b2r = b2.reshape(1, 1)
    logits3, m, s, mn = _mlp_logits(xp, W1, b1r, w2r, b2r)
    logits = logits3.reshape(NPAD)
    logits_col = logits3.reshape(NPAD, 1)

    ar = jnp.arange(BINS, dtype=jnp.int32)
    arf = ar.astype(jnp.float32)
    width1 = (m[0, 0] - mn[0, 0]) / BINS
    edges1 = (mn[0, 0] + arf * width1).reshape(1, BINS)
    suf1 = _suffix_counts(logits_col, edges1)[0]
    b1i = jnp.max(jnp.where(suf1 >= KSEL, ar, 0))
    t1 = edges1[0, b1i]

    width2 = width1 / BINS
    edges2 = (t1 + arf * width2).reshape(1, BINS)
    suf2 = _suffix_counts(logits_col, edges2)[0]
    b2i = jnp.max(jnp.where(suf2 >= KSEL, ar, 0))
    t_edge = edges2[0, b2i].reshape(1, 1)

    dests, mtot = _dests(logits_col, t_edge)
    mtot = jnp.minimum(mtot, MPAD)

    cvals, cidx = _compact_kernel()(logits, dests.reshape(NPAD))
    ar10 = jnp.arange(MPAD, dtype=jnp.int32)
    valid = (ar10 // 160 + 64 * (ar10 % 160)) < mtot[0, 0]
    vals_m = jnp.where(valid, cvals[:MPAD], NEG_INF)
    idx_m = jnp.where(valid, cidx[:MPAD], NPAD + ar10)
    ranks, probs = _rank(vals_m.reshape(MPAD, 1), vals_m.reshape(1, MPAD),
                         idx_m.reshape(MPAD, 1), idx_m.reshape(1, MPAD),
                         m, s)

    locflat = jnp.pad(point_locations, ((0, 8), (0, 0))).reshape(-1)
    outbuf, partials = _final_kernel()(ranks.reshape(MPAD),
                                       probs.reshape(MPAD), idx_m, locflat)
    pmat = partials.reshape(NW, 6, 16).sum(axis=(0, 2))
    mean_location = pmat[0:3] / K
    highest = pmat[3:6]
    return jnp.concatenate([outbuf[:K], mean_location, highest], axis=0)
